# Initial kernel scaffold; baseline (speedup 1.0000x reference)
import jax
import jax.numpy as jnp
from jax.experimental import pallas as pl


def kernel(points_data, pre_gamma, pre_beta, W0, g0, b0, W1, g1, b1, scale_embed, unq_inv):
    raise NotImplementedError("write your pallas kernel here")


# speedup vs baseline: 8.8859x; 8.8859x over previous
import jax
import jax.numpy as jnp
from jax.experimental import pallas as pl
from jax.experimental.pallas import tpu as pltpu

N = 1600000
NUM_VOX = 100000
B = 4000
NB = N // B
PCR = jnp.array([-51.2, -51.2, -5.0], dtype=jnp.float32)
VS = jnp.array([0.1, 0.1, 0.2], dtype=jnp.float32)

_SHIFTS = [2 ** j for j in range(20) if 2 ** j < B]


def _rows(x, k):
    return jnp.concatenate([x[B - k:], x[:B - k]], axis=0)


def _rows_up(x, k):
    return jnp.concatenate([x[k:], x[:k]], axis=0)


def _iota():
    return jax.lax.broadcasted_iota(jnp.int32, (B, 1), 0)


def _seg_prefix(vals, unq, combine):
    it = _iota()
    m = vals
    u = unq
    for k in _SHIFTS:
        cond = (it >= k) & (_rows(u, k) == u)
        m = jnp.where(cond, combine(m, _rows(m, k)), m)
    return m


def _seg_back(vals, unq):
    it = _iota()
    t = vals
    u = unq
    for k in _SHIFTS:
        cond = (it < B - k) & (_rows_up(u, k) == u)
        t = jnp.where(cond, _rows_up(t, k), t)
    return t


def _p1_kernel(pd_ref, unq_ref, out_ref, cval, cunq):
    i = pl.program_id(0)

    @pl.when(i == 0)
    def _():
        cval[...] = jnp.zeros_like(cval)
        cunq[...] = jnp.full_like(cunq, -1)

    pd = pd_ref[...]
    unq = unq_ref[...]
    vals = jnp.concatenate(
        [pd, jnp.ones((B, 1), jnp.float32), jnp.zeros((B, 3), jnp.float32)], axis=1)
    pref = _seg_prefix(vals, unq, jnp.add)
    eq = (unq == cunq[0:1, 0:1]).astype(jnp.float32)
    pref = pref + eq * cval[0:1, 0:8]
    out_ref[...] = pref
    cval[0:1, 0:8] = pref[B - 1:B, :]
    cunq[...] = unq[B - 1:B, 0:1]


def _p2_kernel(pd_ref, unq_ref, pref_ref, feats_ref, st_ref, cval, cunq):
    i = pl.program_id(0)

    @pl.when(i == 0)
    def _():
        cval[...] = jnp.zeros_like(cval)
        cunq[...] = jnp.full_like(cunq, -1)
        st_ref[...] = jnp.zeros_like(st_ref)

    pd = pd_ref[...]
    unq = unq_ref[...]
    t = _seg_back(pref_ref[...], unq)
    eq = unq == cunq[0:1, 0:1]
    t = jnp.where(eq, cval[0:1, 0:8], t)
    cval[0:1, 0:8] = t[0:1, :]
    cunq[...] = unq[0:1, 0:1]

    xyz = pd[:, 0:3]
    mean3 = t[:, 0:3] / jnp.maximum(t[:, 4:5], 1.0)
    f_cluster = xyz - mean3
    fc_cols = []
    for j, (p, v) in enumerate(((-51.2, 0.1), (-51.2, 0.1), (-5.0, 0.2))):
        c = xyz[:, j:j + 1]
        fc_cols.append(c - (jnp.floor((c - p) / v) * v + v / 2.0 + p))
    f_center = jnp.concatenate(fc_cols, axis=1)
    feats = jnp.concatenate(
        [pd, f_cluster, f_center, jnp.zeros((B, 6), jnp.float32)], axis=1)
    feats_ref[...] = feats
    f10 = feats[:, 0:10]

    @pl.when(i == 0)
    def _():
        mu = jnp.sum(f10, axis=0, keepdims=True) / B
        cval[1:2, 0:10] = mu
        st_ref[17:18, 0:10] = mu

    fc = f10 - cval[1:2, 0:10]
    gram = jax.lax.dot_general(fc, fc, (((0,), (0,)), ((), ())),
                               preferred_element_type=jnp.float32, precision=jax.lax.Precision.HIGHEST)
    st_ref[0:16, 0:10] = st_ref[0:16, 0:10] + jnp.pad(gram, ((0, 6), (0, 0)))
    st_ref[16:17, 0:10] = st_ref[16:17, 0:10] + jnp.sum(f10, axis=0, keepdims=True)


def _p3_kernel(feats_ref, unq_ref, a0_ref, c0_ref, out_ref, st_ref, cval, cunq):
    i = pl.program_id(0)

    @pl.when(i == 0)
    def _():
        cval[...] = jnp.full_like(cval, -jnp.inf)
        cunq[...] = jnp.full_like(cunq, -1)
        st_ref[...] = jnp.zeros_like(st_ref)

    unq = unq_ref[...]
    x1 = jnp.maximum(
        jnp.dot(feats_ref[...], a0_ref[...], preferred_element_type=jnp.float32, precision=jax.lax.Precision.HIGHEST)
        + c0_ref[0:1, :], 0.0)
    m = _seg_prefix(x1, unq, jnp.maximum)
    eq = unq == cunq[0:1, 0:1]
    m = jnp.where(eq, jnp.maximum(m, cval[0:1, 0:16]), m)
    out_ref[...] = m
    cval[0:1, 0:16] = m[B - 1:B, :]
    cunq[...] = unq[B - 1:B, 0:1]

    @pl.when(i == 0)
    def _():
        mu = jnp.sum(x1, axis=0, keepdims=True) / B
        cval[1:2, 0:16] = mu
        st_ref[17:18, 0:16] = mu

    xc = x1 - cval[1:2, 0:16]
    gram = jax.lax.dot_general(xc, xc, (((0,), (0,)), ((), ())),
                               preferred_element_type=jnp.float32, precision=jax.lax.Precision.HIGHEST)
    st_ref[0:16, 0:16] = st_ref[0:16, 0:16] + gram
    st_ref[16:17, 0:16] = st_ref[16:17, 0:16] + jnp.sum(x1, axis=0, keepdims=True)


def _p4_kernel(feats_ref, unq_ref, pmax_ref, a0_ref, c0_ref, xg_ref, st_ref,
               cval, cunq):
    i = pl.program_id(0)

    @pl.when(i == 0)
    def _():
        cval[...] = jnp.zeros_like(cval)
        cunq[...] = jnp.full_like(cunq, -1)
        st_ref[...] = jnp.zeros_like(st_ref)

    unq = unq_ref[...]
    t = _seg_back(pmax_ref[...], unq)
    eq = unq == cunq[0:1, 0:1]
    t = jnp.where(eq, cval[0:1, 0:16], t)
    xg_ref[...] = t
    cval[0:1, 0:16] = t[0:1, :]
    cunq[...] = unq[0:1, 0:1]

    x1 = jnp.maximum(
        jnp.dot(feats_ref[...], a0_ref[...], preferred_element_type=jnp.float32, precision=jax.lax.Precision.HIGHEST)
        + c0_ref[0:1, :], 0.0)

    @pl.when(i == 0)
    def _():
        mug = jnp.sum(t, axis=0, keepdims=True) / B
        mux = jnp.sum(x1, axis=0, keepdims=True) / B
        cval[1:2, 0:16] = mug
        cval[2:3, 0:16] = mux
        st_ref[33:34, 0:16] = mug
        st_ref[34:35, 0:16] = mux

    tc = t - cval[1:2, 0:16]
    xc = x1 - cval[2:3, 0:16]
    g1 = jax.lax.dot_general(xc, tc, (((0,), (0,)), ((), ())),
                             preferred_element_type=jnp.float32, precision=jax.lax.Precision.HIGHEST)
    g2 = jax.lax.dot_general(tc, tc, (((0,), (0,)), ((), ())),
                             preferred_element_type=jnp.float32, precision=jax.lax.Precision.HIGHEST)
    st_ref[0:16, 0:16] = st_ref[0:16, 0:16] + g1
    st_ref[16:32, 0:16] = st_ref[16:32, 0:16] + g2
    st_ref[32:33, 0:16] = st_ref[32:33, 0:16] + jnp.sum(t, axis=0, keepdims=True)


def _p5_kernel(feats_ref, xg_ref, unq_ref, a0_ref, c0_ref, a1_ref, c1_ref,
               out_ref, cval, cunq):
    i = pl.program_id(0)

    @pl.when(i == 0)
    def _():
        cval[...] = jnp.full_like(cval, -jnp.inf)
        cunq[...] = jnp.full_like(cunq, -1)

    unq = unq_ref[...]
    x1 = jnp.maximum(
        jnp.dot(feats_ref[...], a0_ref[...], preferred_element_type=jnp.float32, precision=jax.lax.Precision.HIGHEST)
        + c0_ref[0:1, :], 0.0)
    z = jnp.concatenate([x1, xg_ref[...]], axis=1)
    x2 = jnp.maximum(
        jnp.dot(z, a1_ref[...], preferred_element_type=jnp.float32, precision=jax.lax.Precision.HIGHEST)
        + c1_ref[0:1, :], 0.0)
    m = _seg_prefix(x2, unq, jnp.maximum)
    eq = unq == cunq[0:1, 0:1]
    m = jnp.where(eq, jnp.maximum(m, cval[0:1, 0:64]), m)
    out_ref[...] = m
    cval[0:1, 0:64] = m[B - 1:B, :]
    cunq[...] = unq[B - 1:B, 0:1]


def _bspec(bs, fwd=True):
    if fwd:
        return pl.BlockSpec(bs, lambda i: (i, 0))
    return pl.BlockSpec(bs, lambda i: (NB - 1 - i, 0))


def _whole(shape):
    return pl.BlockSpec(shape, lambda i: (0, 0))


def kernel(points_data, pre_gamma, pre_beta, W0, g0, b0, W1, g1, b1,
           scale_embed, unq_inv):
    unq2 = unq_inv.reshape(N, 1)
    nf = jnp.float32(N)

    pref = pl.pallas_call(
        _p1_kernel,
        grid=(NB,),
        in_specs=[_bspec((B, 4)), _bspec((B, 1))],
        out_shape=jax.ShapeDtypeStruct((N, 8), jnp.float32),
        out_specs=_bspec((B, 8)),
        scratch_shapes=[pltpu.VMEM((8, 128), jnp.float32),
                        pltpu.VMEM((1, 1), jnp.int32)],
    )(points_data, unq2)

    feats, st2 = pl.pallas_call(
        _p2_kernel,
        grid=(NB,),
        in_specs=[_bspec((B, 4), False), _bspec((B, 1), False),
                  _bspec((B, 8), False)],
        out_shape=[jax.ShapeDtypeStruct((N, 16), jnp.float32),
                   jax.ShapeDtypeStruct((24, 128), jnp.float32)],
        out_specs=[_bspec((B, 16), False), _whole((24, 128))],
        scratch_shapes=[pltpu.VMEM((8, 128), jnp.float32),
                        pltpu.VMEM((1, 1), jnp.int32)],
    )(points_data, unq2, pref)

    gram_fc = st2[0:10, 0:10]
    sum_f = st2[16, 0:10]
    mu0f = st2[17, 0:10]
    m1 = sum_f / nf
    dmf = m1 - mu0f
    cov_f = gram_fc / nf - jnp.outer(dmf, dmf)
    v1 = jnp.diag(cov_f)
    s1 = pre_gamma / jnp.sqrt(v1 + 1e-5)
    t1 = pre_beta - m1 * s1

    cov_fn = s1[:, None] * cov_f * s1[None, :]
    m2 = (m1 * s1 + t1) @ W0
    v2 = jnp.einsum("ij,ik,kj->j", W0, cov_fn, W0)
    s2 = g0 / jnp.sqrt(v2 + 1e-3)
    t2 = b0 - m2 * s2
    a0 = jnp.pad(s1[:, None] * W0 * s2[None, :], ((0, 6), (0, 0)))
    c0 = ((t1 @ W0) * s2 + t2).reshape(1, 16)

    pmax, st3 = pl.pallas_call(
        _p3_kernel,
        grid=(NB,),
        in_specs=[_bspec((B, 16)), _bspec((B, 1)), _whole((16, 16)),
                  _whole((1, 16))],
        out_shape=[jax.ShapeDtypeStruct((N, 16), jnp.float32),
                   jax.ShapeDtypeStruct((24, 128), jnp.float32)],
        out_specs=[_bspec((B, 16)), _whole((24, 128))],
        scratch_shapes=[pltpu.VMEM((8, 128), jnp.float32),
                        pltpu.VMEM((1, 1), jnp.int32)],
    )(feats, unq2, a0, c0)

    xg, st4 = pl.pallas_call(
        _p4_kernel,
        grid=(NB,),
        in_specs=[_bspec((B, 16), False), _bspec((B, 1), False),
                  _bspec((B, 16), False), _whole((16, 16)), _whole((1, 16))],
        out_shape=[jax.ShapeDtypeStruct((N, 16), jnp.float32),
                   jax.ShapeDtypeStruct((40, 128), jnp.float32)],
        out_specs=[_bspec((B, 16), False), _whole((40, 128))],
        scratch_shapes=[pltpu.VMEM((8, 128), jnp.float32),
                        pltpu.VMEM((1, 1), jnp.int32)],
    )(feats, unq2, pmax, a0, c0)

    m_x1 = st3[16, 0:16] / nf
    mu3 = st3[17, 0:16]
    d3 = m_x1 - mu3
    c11 = st3[0:16, 0:16] / nf - jnp.outer(d3, d3)
    m_xg = st4[32, 0:16] / nf
    mug4 = st4[33, 0:16]
    mux4 = st4[34, 0:16]
    dg = m_xg - mug4
    dx = m_x1 - mux4
    c1g = st4[0:16, 0:16] / nf - jnp.outer(dx, dg)
    cgg = st4[16:32, 0:16] / nf - jnp.outer(dg, dg)
    m_z = jnp.concatenate([m_x1, m_xg])
    cov_z = jnp.block([[c11, c1g], [c1g.T, cgg]])
    m3 = m_z @ W1
    v3 = jnp.einsum("ij,ik,kj->j", W1, cov_z, W1)
    s3 = g1 / jnp.sqrt(v3 + 1e-3)
    t3 = b1 - m3 * s3
    a1 = W1 * s3[None, :]
    c1 = t3.reshape(1, 64)

    pm2 = pl.pallas_call(
        _p5_kernel,
        grid=(NB,),
        in_specs=[_bspec((B, 16)), _bspec((B, 16)), _bspec((B, 1)),
                  _whole((16, 16)), _whole((1, 16)), _whole((32, 64)),
                  _whole((1, 64))],
        out_shape=jax.ShapeDtypeStruct((N, 64), jnp.float32),
        out_specs=_bspec((B, 64)),
        scratch_shapes=[pltpu.VMEM((8, 128), jnp.float32),
                        pltpu.VMEM((1, 1), jnp.int32)],
    )(feats, xg, unq2, a0, c0, a1, c1)

    vox = jnp.arange(NUM_VOX, dtype=jnp.int32)
    right = jnp.searchsorted(unq_inv, vox, side="right")
    left = jnp.searchsorted(unq_inv, vox, side="left")
    nonempty = (right > left)[:, None]
    idx = jnp.clip(right - 1, 0, N - 1)
    out = jnp.where(nonempty, pm2[idx], 0.0) + scale_embed[0][None, :]
    return out
